# Initial kernel scaffold; baseline (speedup 1.0000x reference)
import functools

import jax
import jax.numpy as jnp
from jax import lax
from jax.experimental import pallas as pl
from jax.experimental.pallas import tpu as pltpu
from jax.experimental.pallas import tpu_sc as plsc

B = 16384
CAT_EMB = 64
SUBCAT_EMB = 128
OUT_DIM = 128
NUM_SCALARS = 8

_NC = 2
_NS = 16
_NW = _NC * _NS
_BPW = B // _NW
_CHUNK = 128
_NCH = _BPW // _CHUNK
_IDX_ROWS = B // _CHUNK


def _build_sc_gather():
    mesh = plsc.VectorSubcoreMesh(core_axis_name="c", subcore_axis_name="s")

    @functools.partial(
        pl.kernel,
        mesh=mesh,
        out_type=[
            jax.ShapeDtypeStruct((B, SUBCAT_EMB), jnp.float32),
            jax.ShapeDtypeStruct((B, CAT_EMB), jnp.float32),
        ],
        scratch_types=[
            pltpu.VMEM((_NCH, _CHUNK), jnp.int32),
            pltpu.VMEM((_NCH, _CHUNK), jnp.int32),
            pltpu.VMEM((_BPW, SUBCAT_EMB), jnp.float32),
            pltpu.VMEM((_BPW, CAT_EMB), jnp.float32),
            pltpu.SemaphoreType.DMA,
            pltpu.SemaphoreType.DMA,
        ],
    )
    def gather_kernel(sub_idx_hbm, cat_idx_hbm, sub_tbl_hbm, cat_tbl_hbm,
                      se_hbm, ce_hbm, sidx, cidx, srows, crows, ssem, csem):
        wid = lax.axis_index("s") * _NC + lax.axis_index("c")
        idx_row0 = wid * _NCH
        pltpu.sync_copy(sub_idx_hbm.at[pl.ds(idx_row0, _NCH)], sidx)
        pltpu.sync_copy(cat_idx_hbm.at[pl.ds(idx_row0, _NCH)], cidx)
        copies = []
        for j in range(_NCH):
            copies.append(pltpu.async_copy(
                sub_tbl_hbm.at[sidx.at[j]],
                srows.at[pl.ds(j * _CHUNK, _CHUNK)], ssem))
            copies.append(pltpu.async_copy(
                cat_tbl_hbm.at[cidx.at[j]],
                crows.at[pl.ds(j * _CHUNK, _CHUNK)], csem))
        for c in copies:
            c.wait()
        base = wid * _BPW
        pltpu.sync_copy(srows, se_hbm.at[pl.ds(base, _BPW)])
        pltpu.sync_copy(crows, ce_hbm.at[pl.ds(base, _BPW)])

    return gather_kernel


_sc_gather = _build_sc_gather()

_BB = 2048


def _mlp_body(ce_ref, se_ref, s_ref, wc_ref, ws_ref, wr_ref, b_ref, o_ref):
    s = s_ref[...]
    col = lax.broadcasted_iota(jnp.int32, s.shape, 1)
    s = jnp.where(col == 0, jnp.log1p(s), s)
    h = jnp.dot(ce_ref[...], wc_ref[...], preferred_element_type=jnp.float32)
    h = h + jnp.dot(se_ref[...], ws_ref[...], preferred_element_type=jnp.float32)
    h = h + jnp.dot(s, wr_ref[...], preferred_element_type=jnp.float32)
    h = h + b_ref[...]
    o_ref[...] = 0.5 * h * (1.0 + lax.erf(h * 0.7071067811865476))


@jax.jit
def kernel(category_idx, subcategory_idx, total_weight, step_zscore,
           stage_coverage, mask_flags, cat_table, subcat_table, W, b):
    sub_idx = subcategory_idx.astype(jnp.int32).reshape(_IDX_ROWS, _CHUNK)
    cat_idx = category_idx.astype(jnp.int32).reshape(_IDX_ROWS, _CHUNK)
    se, ce = _sc_gather(sub_idx, cat_idx, subcat_table, cat_table)

    s = jnp.concatenate(
        [total_weight[:, None], step_zscore[:, None], stage_coverage[:, None],
         mask_flags], axis=1)
    wc = W[:CAT_EMB]
    ws = W[CAT_EMB:CAT_EMB + SUBCAT_EMB]
    wr = W[CAT_EMB + SUBCAT_EMB:]
    b2 = b[None, :]

    out = pl.pallas_call(
        _mlp_body,
        grid=(B // _BB,),
        in_specs=[
            pl.BlockSpec((_BB, CAT_EMB), lambda i: (i, 0)),
            pl.BlockSpec((_BB, SUBCAT_EMB), lambda i: (i, 0)),
            pl.BlockSpec((_BB, NUM_SCALARS), lambda i: (i, 0)),
            pl.BlockSpec((CAT_EMB, OUT_DIM), lambda i: (0, 0)),
            pl.BlockSpec((SUBCAT_EMB, OUT_DIM), lambda i: (0, 0)),
            pl.BlockSpec((NUM_SCALARS, OUT_DIM), lambda i: (0, 0)),
            pl.BlockSpec((1, OUT_DIM), lambda i: (0, 0)),
        ],
        out_specs=pl.BlockSpec((_BB, OUT_DIM), lambda i: (i, 0)),
        out_shape=jax.ShapeDtypeStruct((B, OUT_DIM), jnp.float32),
    )(ce, se, s, wc, ws, wr, b2)
    return out


# speedup vs baseline: 3.3721x; 3.3721x over previous
import functools

import jax
import jax.numpy as jnp
from jax import lax
from jax.experimental import pallas as pl
from jax.experimental.pallas import tpu as pltpu
from jax.experimental.pallas import tpu_sc as plsc

B = 16384
VOCAB_CAT = 1000
CAT_EMB = 64
SUBCAT_EMB = 128
OUT_DIM = 128
NUM_SCALARS = 8

_NC = 2
_NS = 16
_NW = _NC * _NS
_BPW = B // _NW
_CHUNK = 128
_NCH = _BPW // _CHUNK
_IDX_ROWS = B // _CHUNK


@functools.lru_cache(maxsize=None)
def _build_sc_gather():
    mesh = plsc.VectorSubcoreMesh(core_axis_name="c", subcore_axis_name="s")

    @functools.partial(
        pl.kernel,
        mesh=mesh,
        out_type=[
            jax.ShapeDtypeStruct((B, SUBCAT_EMB), jnp.float32),
            jax.ShapeDtypeStruct((B, OUT_DIM), jnp.float32),
        ],
        scratch_types=[
            pltpu.VMEM((_NCH, _CHUNK), jnp.int32),
            pltpu.VMEM((_NCH, _CHUNK), jnp.int32),
            pltpu.VMEM((_BPW // 2, SUBCAT_EMB), jnp.float32),
            pltpu.VMEM((_BPW // 2, OUT_DIM), jnp.float32),
            pltpu.SemaphoreType.DMA,
            pltpu.SemaphoreType.DMA,
        ],
    )
    def gather_kernel(sub_idx_hbm, cat_idx_hbm, sub_tbl_hbm, cat_tbl_hbm,
                      se_hbm, ce_hbm, sidx, cidx, srows, crows, ssem, csem):
        wid = lax.axis_index("s") * _NC + lax.axis_index("c")
        idx_row0 = wid * _NCH
        pltpu.sync_copy(sub_idx_hbm.at[pl.ds(idx_row0, _NCH)], sidx)
        pltpu.sync_copy(cat_idx_hbm.at[pl.ds(idx_row0, _NCH)], cidx)
        base = wid * _BPW
        half = _BPW // 2
        for p in range(2):
            copies = []
            for j in range(_NCH // 2):
                ch = p * (_NCH // 2) + j
                copies.append(pltpu.async_copy(
                    sub_tbl_hbm.at[sidx.at[ch]],
                    srows.at[pl.ds(j * _CHUNK, _CHUNK)], ssem))
                copies.append(pltpu.async_copy(
                    cat_tbl_hbm.at[cidx.at[ch]],
                    crows.at[pl.ds(j * _CHUNK, _CHUNK)], csem))
            for c in copies:
                c.wait()
            pltpu.sync_copy(srows, se_hbm.at[pl.ds(base + p * half, half)])
            pltpu.sync_copy(crows, ce_hbm.at[pl.ds(base + p * half, half)])

    return gather_kernel


def _fuse_cat_body(tbl_ref, w_ref, b_ref, o_ref):
    o_ref[...] = jnp.dot(tbl_ref[...], w_ref[...],
                         preferred_element_type=jnp.float32) + b_ref[...]


_BB = 2048


def _mlp_body(cef_ref, se_ref, s_ref, ws_ref, wr_ref, o_ref):
    s = s_ref[...]
    col = lax.broadcasted_iota(jnp.int32, s.shape, 1)
    s = jnp.where(col == 0, jnp.log1p(s), s)
    h = cef_ref[...]
    h = h + jnp.dot(se_ref[...], ws_ref[...], preferred_element_type=jnp.float32)
    h = h + jnp.dot(s, wr_ref[...], preferred_element_type=jnp.float32)
    o_ref[...] = 0.5 * h * (1.0 + lax.erf(h * 0.7071067811865476))


@jax.jit
def kernel(category_idx, subcategory_idx, total_weight, step_zscore,
           stage_coverage, mask_flags, cat_table, subcat_table, W, b):
    fused_cat = pl.pallas_call(
        _fuse_cat_body,
        in_specs=[
            pl.BlockSpec((VOCAB_CAT, CAT_EMB), lambda: (0, 0)),
            pl.BlockSpec((CAT_EMB, OUT_DIM), lambda: (0, 0)),
            pl.BlockSpec((1, OUT_DIM), lambda: (0, 0)),
        ],
        out_specs=pl.BlockSpec((VOCAB_CAT, OUT_DIM), lambda: (0, 0)),
        out_shape=jax.ShapeDtypeStruct((VOCAB_CAT, OUT_DIM), jnp.float32),
    )(cat_table, W[:CAT_EMB], b[None, :])

    sub_idx = subcategory_idx.astype(jnp.int32).reshape(_IDX_ROWS, _CHUNK)
    cat_idx = category_idx.astype(jnp.int32).reshape(_IDX_ROWS, _CHUNK)
    se, cef = _build_sc_gather()(sub_idx, cat_idx, subcat_table, fused_cat)

    s = jnp.concatenate(
        [total_weight[:, None], step_zscore[:, None], stage_coverage[:, None],
         mask_flags], axis=1)
    ws = W[CAT_EMB:CAT_EMB + SUBCAT_EMB]
    wr = W[CAT_EMB + SUBCAT_EMB:]

    out = pl.pallas_call(
        _mlp_body,
        grid=(B // _BB,),
        in_specs=[
            pl.BlockSpec((_BB, OUT_DIM), lambda i: (i, 0)),
            pl.BlockSpec((_BB, SUBCAT_EMB), lambda i: (i, 0)),
            pl.BlockSpec((_BB, NUM_SCALARS), lambda i: (i, 0)),
            pl.BlockSpec((SUBCAT_EMB, OUT_DIM), lambda i: (0, 0)),
            pl.BlockSpec((NUM_SCALARS, OUT_DIM), lambda i: (0, 0)),
        ],
        out_specs=pl.BlockSpec((_BB, OUT_DIM), lambda i: (i, 0)),
        out_shape=jax.ShapeDtypeStruct((B, OUT_DIM), jnp.float32),
    )(cef, se, s, ws, wr)
    return out


# speedup vs baseline: 3.8219x; 1.1334x over previous
import functools

import jax
import jax.numpy as jnp
from jax import lax
from jax.experimental import pallas as pl
from jax.experimental.pallas import tpu as pltpu
from jax.experimental.pallas import tpu_sc as plsc

B = 16384
VOCAB_CAT = 1000
CAT_EMB = 64
SUBCAT_EMB = 128
OUT_DIM = 128
NUM_SCALARS = 8

_NC = 2
_NS = 16
_NW = _NC * _NS
_BPW = B // _NW
_CHUNK = 128
_NCH = _BPW // _CHUNK
_IDX_ROWS = B // _CHUNK


@functools.lru_cache(maxsize=None)
def _build_sc_gather():
    mesh = plsc.VectorSubcoreMesh(core_axis_name="c", subcore_axis_name="s")

    @functools.partial(
        pl.kernel,
        mesh=mesh,
        out_type=[
            jax.ShapeDtypeStruct((B, SUBCAT_EMB), jnp.float32),
            jax.ShapeDtypeStruct((B, OUT_DIM), jnp.float32),
        ],
        scratch_types=[
            pltpu.VMEM((_NCH, _CHUNK), jnp.int32),
            pltpu.VMEM((_NCH, _CHUNK), jnp.int32),
            pltpu.VMEM((_BPW // 2, SUBCAT_EMB), jnp.float32),
            pltpu.VMEM((_BPW // 2, OUT_DIM), jnp.float32),
            pltpu.SemaphoreType.DMA,
            pltpu.SemaphoreType.DMA,
        ],
    )
    def gather_kernel(sub_idx_hbm, cat_idx_hbm, sub_tbl_hbm, cat_tbl_hbm,
                      se_hbm, ce_hbm, sidx, cidx, srows, crows, ssem, csem):
        wid = lax.axis_index("s") * _NC + lax.axis_index("c")
        idx_row0 = wid * _NCH
        pltpu.sync_copy(sub_idx_hbm.at[pl.ds(idx_row0, _NCH)], sidx)
        pltpu.sync_copy(cat_idx_hbm.at[pl.ds(idx_row0, _NCH)], cidx)
        base = wid * _BPW
        half = _BPW // 2
        for p in range(2):
            copies = []
            for j in range(_NCH // 2):
                ch = p * (_NCH // 2) + j
                copies.append(pltpu.async_copy(
                    sub_tbl_hbm.at[sidx.at[ch]],
                    srows.at[pl.ds(j * _CHUNK, _CHUNK)], ssem))
                copies.append(pltpu.async_copy(
                    cat_tbl_hbm.at[cidx.at[ch]],
                    crows.at[pl.ds(j * _CHUNK, _CHUNK)], csem))
            for c in copies:
                c.wait()
            pltpu.sync_copy(srows, se_hbm.at[pl.ds(base + p * half, half)])
            pltpu.sync_copy(crows, ce_hbm.at[pl.ds(base + p * half, half)])

    return gather_kernel


def _fuse_cat_body(tblT_ref, w_ref, b_ref, o_ref):
    o_ref[...] = lax.dot_general(
        tblT_ref[...], w_ref[...], (((0,), (0,)), ((), ())),
        preferred_element_type=jnp.float32) + b_ref[...]


_BB = 2048


def _mlp_body(cef_ref, se_ref, sT_ref, ws_ref, wr_ref, o_ref):
    sT = sT_ref[...]
    row = lax.broadcasted_iota(jnp.int32, sT.shape, 0)
    sT = jnp.where(row == 0, jnp.log1p(sT), sT)
    h = cef_ref[...]
    h = h + jnp.dot(se_ref[...], ws_ref[...], preferred_element_type=jnp.float32)
    h = h + lax.dot_general(sT, wr_ref[...], (((0,), (0,)), ((), ())),
                            preferred_element_type=jnp.float32)
    o_ref[...] = 0.5 * h * (1.0 + lax.erf(h * 0.7071067811865476))


@jax.jit
def kernel(category_idx, subcategory_idx, total_weight, step_zscore,
           stage_coverage, mask_flags, cat_table, subcat_table, W, b):
    fused_cat = pl.pallas_call(
        _fuse_cat_body,
        in_specs=[
            pl.BlockSpec((CAT_EMB, VOCAB_CAT), lambda: (0, 0)),
            pl.BlockSpec((CAT_EMB, OUT_DIM), lambda: (0, 0)),
            pl.BlockSpec((1, OUT_DIM), lambda: (0, 0)),
        ],
        out_specs=pl.BlockSpec((VOCAB_CAT, OUT_DIM), lambda: (0, 0)),
        out_shape=jax.ShapeDtypeStruct((VOCAB_CAT, OUT_DIM), jnp.float32),
    )(cat_table.T, W[:CAT_EMB], b[None, :])

    sub_idx = subcategory_idx.astype(jnp.int32).reshape(_IDX_ROWS, _CHUNK)
    cat_idx = category_idx.astype(jnp.int32).reshape(_IDX_ROWS, _CHUNK)
    se, cef = _build_sc_gather()(sub_idx, cat_idx, subcat_table, fused_cat)

    sT = jnp.concatenate(
        [total_weight[None, :], step_zscore[None, :], stage_coverage[None, :],
         mask_flags.T], axis=0)
    ws = W[CAT_EMB:CAT_EMB + SUBCAT_EMB]
    wr = W[CAT_EMB + SUBCAT_EMB:]

    out = pl.pallas_call(
        _mlp_body,
        grid=(B // _BB,),
        in_specs=[
            pl.BlockSpec((_BB, OUT_DIM), lambda i: (i, 0)),
            pl.BlockSpec((_BB, SUBCAT_EMB), lambda i: (i, 0)),
            pl.BlockSpec((NUM_SCALARS, _BB), lambda i: (0, i)),
            pl.BlockSpec((SUBCAT_EMB, OUT_DIM), lambda i: (0, 0)),
            pl.BlockSpec((NUM_SCALARS, OUT_DIM), lambda i: (0, 0)),
        ],
        out_specs=pl.BlockSpec((_BB, OUT_DIM), lambda i: (i, 0)),
        out_shape=jax.ShapeDtypeStruct((B, OUT_DIM), jnp.float32),
    )(cef, se, sT, ws, wr)
    return out
